# baseline (speedup 1.0000x reference)
import jax
import jax.numpy as jnp
from jax.experimental import pallas as pl

_F32 = jnp.float32
_BF16 = jnp.bfloat16
_HPS = 3
_CHUNK = 1024


def _fused_body(feat_ref, pos_ref, mask_ref, wq_ref, bq_ref, wk_ref, bk_ref,
                wpos_ref, sel_ref, wv_ref, bv_ref, wo_ref, bo_ref, out_ref):
    h = pl.program_id(0)
    f = feat_ref[...]
    n = f.shape[0]
    p = pos_ref[...]
    mx = jnp.max(p, axis=0, keepdims=True)
    pn = p / jnp.maximum(mx, 1e-30)
    d = p.shape[1]
    mask_term = mask_ref[...] * 100.0

    def head(i):
        q = jnp.dot(f, wq_ref[i], preferred_element_type=_F32) + bq_ref[i]
        w_pos = wpos_ref[i]
        bias_cols = pn[:, 0:1] * w_pos[0:1, :]
        for j in range(1, d):
            bias_cols = bias_cols + pn[:, j:j + 1] * w_pos[j:j + 1, :]
        bias_cols = bias_cols + mask_term * sel_ref[i]
        kk = (jnp.dot(f, wk_ref[i], preferred_element_type=_F32)
              + bias_cols + bk_ref[i])
        vv = (jnp.dot(f, wv_ref[i], preferred_element_type=_F32)
              + bv_ref[i]).astype(_BF16)
        return q, kk, vv

    hps = wq_ref.shape[0]
    qkv = [head(i) for i in range(hps)]

    ch = _CHUNK
    for ci in range(n // ch):
        rows = slice(ci * ch, (ci + 1) * ch)
        ss = [jax.lax.dot_general(q[rows], kk, (((1,), (1,)), ((), ())),
                                  preferred_element_type=_F32)
              for q, kk, _ in qkv]
        es = [jnp.exp(s - jnp.max(s, axis=-1, keepdims=True)) for s in ss]
        sums = [jnp.sum(e, axis=-1, keepdims=True) for e in es]
        es = [e.astype(_BF16) for e in es]
        os_ = [jnp.dot(e, v, preferred_element_type=_F32)
               for e, (_, _, v) in zip(es, qkv)]
        os_ = [o * (1.0 / r) for o, r in zip(os_, sums)]
        contrib = jnp.dot(os_[0], wo_ref[0], preferred_element_type=_F32)
        for i in range(1, hps):
            contrib = contrib + jnp.dot(os_[i], wo_ref[i],
                                        preferred_element_type=_F32)

        @pl.when(h == 0)
        def _(contrib=contrib, rows=rows):
            out_ref[rows, :] = contrib + bo_ref[...]

        @pl.when(h != 0)
        def _(contrib=contrib, rows=rows):
            out_ref[rows, :] = out_ref[rows, :] + contrib


def kernel(pos, feat, mask, k, pos_lambda, qkv_w, qkv_b, pos_mlp_w,
           pos_mlp_b, proj_w, proj_b):
    b, n, c = feat.shape
    d = pos.shape[2]
    nh = pos_mlp_w.shape[0]
    c_ = c // nh
    qd = c_ // 2
    vd = 2 * c_
    AUG = 64
    scale = (c_ ** -0.5) * k

    feat2 = feat.reshape(n, c)
    pos2 = pos.reshape(n, d).astype(_F32)
    mask2 = mask.reshape(n, 1)

    w6 = qkv_w.reshape(nh, 6, qd, c)
    b6 = qkv_b.reshape(nh, 6, qd)
    wq = jnp.swapaxes(w6[:, 0], 1, 2)
    wk = jnp.swapaxes(w6[:, 1], 1, 2)
    wv = jnp.swapaxes(w6[:, 2:].reshape(nh, vd, c), 1, 2)
    bq, bk = b6[:, 0], b6[:, 1]
    bv = b6[:, 2:].reshape(nh, 1, vd)

    wq_aug = jnp.zeros((nh, c, AUG), _F32).at[:, :, :qd].set(wq * scale)
    bq_aug = (jnp.zeros((nh, 1, AUG), _F32).at[:, 0, :qd].set(bq * scale)
              .at[:, 0, qd].set(1.0))
    wk_aug = jnp.zeros((nh, c, AUG), _F32).at[:, :, :qd].set(wk)
    bk_aug = (jnp.zeros((nh, 1, AUG), _F32).at[:, 0, :qd].set(bk)
              .at[:, 0, qd].set(-100.0))
    wpos_aug = jnp.zeros((nh, d, AUG), _F32).at[:, :, qd].set(pos_mlp_w[:, :, 0])
    sel_aug = jnp.zeros((nh, 1, AUG), _F32).at[:, 0, qd].set(1.0)

    wo = jnp.transpose(proj_w.reshape(c, nh, vd), (1, 2, 0))
    bo = proj_b.reshape(1, c)

    hps = _HPS
    out2 = pl.pallas_call(
        _fused_body,
        grid=(nh // hps,),
        in_specs=[
            pl.BlockSpec((n, c), lambda hh: (0, 0)),
            pl.BlockSpec((n, d), lambda hh: (0, 0)),
            pl.BlockSpec((n, 1), lambda hh: (0, 0)),
            pl.BlockSpec((hps, c, AUG), lambda hh: (hh, 0, 0)),
            pl.BlockSpec((hps, 1, AUG), lambda hh: (hh, 0, 0)),
            pl.BlockSpec((hps, c, AUG), lambda hh: (hh, 0, 0)),
            pl.BlockSpec((hps, 1, AUG), lambda hh: (hh, 0, 0)),
            pl.BlockSpec((hps, d, AUG), lambda hh: (hh, 0, 0)),
            pl.BlockSpec((hps, 1, AUG), lambda hh: (hh, 0, 0)),
            pl.BlockSpec((hps, c, vd), lambda hh: (hh, 0, 0)),
            pl.BlockSpec((hps, 1, vd), lambda hh: (hh, 0, 0)),
            pl.BlockSpec((hps, vd, c), lambda hh: (hh, 0, 0)),
            pl.BlockSpec((1, c), lambda hh: (0, 0)),
        ],
        out_specs=pl.BlockSpec((n, c), lambda hh: (0, 0)),
        out_shape=jax.ShapeDtypeStruct((n, c), _F32),
    )(feat2, pos2, mask2, wq_aug, bq_aug, wk_aug, bk_aug, wpos_aug, sel_aug,
      wv, bv, wo, bo)

    return out2.reshape(b, n, c)


# speedup vs baseline: 1.0604x; 1.0604x over previous
import jax
import jax.numpy as jnp
from jax.experimental import pallas as pl

_F32 = jnp.float32
_HPS = 3
_CHUNK = 1024


def _fused_body(feat_ref, pos_ref, mask_ref, wq_ref, bq_ref, wk_ref, bk_ref,
                wpos_ref, sel_ref, wv_ref, bv_ref, wo_ref, bo_ref, out_ref):
    h = pl.program_id(0)
    f = feat_ref[...]
    n = f.shape[0]
    p = pos_ref[...]
    mx = jnp.max(p, axis=0, keepdims=True)
    pn = p / jnp.maximum(mx, 1e-30)
    d = p.shape[1]
    mask_term = mask_ref[...] * 100.0

    def head(i):
        q = jnp.dot(f, wq_ref[i], preferred_element_type=_F32) + bq_ref[i]
        w_pos = wpos_ref[i]
        bias_cols = pn[:, 0:1] * w_pos[0:1, :]
        for j in range(1, d):
            bias_cols = bias_cols + pn[:, j:j + 1] * w_pos[j:j + 1, :]
        bias_cols = bias_cols + mask_term * sel_ref[i]
        kk = (jnp.dot(f, wk_ref[i], preferred_element_type=_F32)
              + bias_cols + bk_ref[i])
        vv = jnp.dot(f, wv_ref[i], preferred_element_type=_F32) + bv_ref[i]
        return q, kk, vv

    hps = wq_ref.shape[0]
    qkv = [head(i) for i in range(hps)]

    ch = _CHUNK
    for ci in range(n // ch):
        rows = slice(ci * ch, (ci + 1) * ch)
        ss = [jax.lax.dot_general(q[rows], kk, (((1,), (1,)), ((), ())),
                                  preferred_element_type=_F32)
              for q, kk, _ in qkv]
        es = [jnp.exp(s - jnp.max(s, axis=-1, keepdims=True)) for s in ss]
        os_ = [jnp.dot(e, v, preferred_element_type=_F32)
               for e, (_, _, v) in zip(es, qkv)]
        os_ = [o * (1.0 / jnp.sum(e, axis=-1, keepdims=True))
               for o, e in zip(os_, es)]
        contrib = jnp.dot(os_[0], wo_ref[0], preferred_element_type=_F32)
        for i in range(1, hps):
            contrib = contrib + jnp.dot(os_[i], wo_ref[i],
                                        preferred_element_type=_F32)

        @pl.when(h == 0)
        def _(contrib=contrib, rows=rows):
            out_ref[rows, :] = contrib + bo_ref[...]

        @pl.when(h != 0)
        def _(contrib=contrib, rows=rows):
            out_ref[rows, :] = out_ref[rows, :] + contrib


def kernel(pos, feat, mask, k, pos_lambda, qkv_w, qkv_b, pos_mlp_w,
           pos_mlp_b, proj_w, proj_b):
    b, n, c = feat.shape
    d = pos.shape[2]
    nh = pos_mlp_w.shape[0]
    c_ = c // nh
    qd = c_ // 2
    vd = 2 * c_
    AUG = 64
    scale = (c_ ** -0.5) * k

    feat2 = feat.reshape(n, c)
    pos2 = pos.reshape(n, d).astype(_F32)
    mask2 = mask.reshape(n, 1)

    w6 = qkv_w.reshape(nh, 6, qd, c)
    b6 = qkv_b.reshape(nh, 6, qd)
    wq = jnp.swapaxes(w6[:, 0], 1, 2)
    wk = jnp.swapaxes(w6[:, 1], 1, 2)
    wv = jnp.swapaxes(w6[:, 2:].reshape(nh, vd, c), 1, 2)
    bq, bk = b6[:, 0], b6[:, 1]
    bv = b6[:, 2:].reshape(nh, 1, vd)

    wq_aug = jnp.zeros((nh, c, AUG), _F32).at[:, :, :qd].set(wq * scale)
    bq_aug = (jnp.zeros((nh, 1, AUG), _F32).at[:, 0, :qd].set(bq * scale)
              .at[:, 0, qd].set(1.0))
    wk_aug = jnp.zeros((nh, c, AUG), _F32).at[:, :, :qd].set(wk)
    bk_aug = (jnp.zeros((nh, 1, AUG), _F32).at[:, 0, :qd].set(bk)
              .at[:, 0, qd].set(-100.0))
    wpos_aug = jnp.zeros((nh, d, AUG), _F32).at[:, :, qd].set(pos_mlp_w[:, :, 0])
    sel_aug = jnp.zeros((nh, 1, AUG), _F32).at[:, 0, qd].set(1.0)

    wo = jnp.transpose(proj_w.reshape(c, nh, vd), (1, 2, 0))
    bo = proj_b.reshape(1, c)

    hps = _HPS
    out2 = pl.pallas_call(
        _fused_body,
        grid=(nh // hps,),
        in_specs=[
            pl.BlockSpec((n, c), lambda hh: (0, 0)),
            pl.BlockSpec((n, d), lambda hh: (0, 0)),
            pl.BlockSpec((n, 1), lambda hh: (0, 0)),
            pl.BlockSpec((hps, c, AUG), lambda hh: (hh, 0, 0)),
            pl.BlockSpec((hps, 1, AUG), lambda hh: (hh, 0, 0)),
            pl.BlockSpec((hps, c, AUG), lambda hh: (hh, 0, 0)),
            pl.BlockSpec((hps, 1, AUG), lambda hh: (hh, 0, 0)),
            pl.BlockSpec((hps, d, AUG), lambda hh: (hh, 0, 0)),
            pl.BlockSpec((hps, 1, AUG), lambda hh: (hh, 0, 0)),
            pl.BlockSpec((hps, c, vd), lambda hh: (hh, 0, 0)),
            pl.BlockSpec((hps, 1, vd), lambda hh: (hh, 0, 0)),
            pl.BlockSpec((hps, vd, c), lambda hh: (hh, 0, 0)),
            pl.BlockSpec((1, c), lambda hh: (0, 0)),
        ],
        out_specs=pl.BlockSpec((n, c), lambda hh: (0, 0)),
        out_shape=jax.ShapeDtypeStruct((n, c), _F32),
    )(feat2, pos2, mask2, wq_aug, bq_aug, wk_aug, bk_aug, wpos_aug, sel_aug,
      wv, bv, wo, bo)

    return out2.reshape(b, n, c)
